# baseline (speedup 1.0000x reference)
import functools

import jax
import jax.numpy as jnp
from jax import lax
from jax.experimental import pallas as pl
from jax.experimental.pallas import tpu as pltpu
from jax.experimental.pallas import tpu_sc as plsc

_NC, _NS = 2, 16
_NW = _NC * _NS
_CHUNK = 16 * 1024
_LANES = 16


_ROWS = 8
_D = 1024
_B = 4


def _sc_body(x_hbm, p_hbm, o_hbm, xb, pb, sx, sp, so):
    S = p_hbm.shape[0]
    spw = S // _NW
    nstep = spw // _ROWS
    wid = lax.axis_index("s") * _NC + lax.axis_index("c")
    s_base = wid * spw

    def in_copies(step, slot):
        s0 = s_base + step * _ROWS
        cps = [pltpu.make_async_copy(
            p_hbm.at[pl.ds(s0, _ROWS)], pb.at[slot], sp.at[slot])]
        for b in range(_B):
            cps.append(pltpu.make_async_copy(
                x_hbm.at[pl.ds(b * S + s0, _ROWS)], xb.at[slot, b], sx.at[slot]))
        return cps

    def out_copies(step, slot):
        s0 = s_base + step * _ROWS
        return [pltpu.make_async_copy(
            xb.at[slot, b], o_hbm.at[pl.ds(b * S + s0, _ROWS)], so.at[slot])
            for b in range(_B)]

    def compute(slot):
        @plsc.parallel_loop(0, _ROWS * _D, step=_LANES, unroll=8)
        def _(off):
            r = off // _D
            c = off % _D
            pv = pb[slot, r, pl.ds(c, _LANES)]
            for b in range(_B):
                plsc.addupdate(xb.at[slot, b, r, pl.ds(c, _LANES)], pv)

    for c in in_copies(0, 0):
        c.start()
    for s in range(nstep):
        slot = s % 2
        if s + 1 < nstep:
            if s >= 1:
                for c in out_copies(s - 1, 1 - slot):
                    c.wait()
            for c in in_copies(s + 1, 1 - slot):
                c.start()
        for c in in_copies(s, slot):
            c.wait()
        compute(slot)
        for c in out_copies(s, slot):
            c.start()
    if nstep >= 2:
        for c in out_copies(nstep - 2, (nstep - 2) % 2):
            c.wait()
    for c in out_copies(nstep - 1, (nstep - 1) % 2):
        c.wait()


def _sc_add(x, pos_table):
    B, S, D = x.shape
    xf = x.reshape(B * S, D)
    pf = pos_table
    run = pl.kernel(
        _sc_body,
        out_type=jax.ShapeDtypeStruct((B * S, D), x.dtype),
        mesh=plsc.VectorSubcoreMesh(
            core_axis_name="c", subcore_axis_name="s",
            num_cores=_NC, num_subcores=_NS,
        ),
        scratch_types=[
            pltpu.VMEM((2, _B, _ROWS, _D), jnp.float32),
            pltpu.VMEM((2, _ROWS, _D), jnp.float32),
            pltpu.SemaphoreType.DMA((2,)),
            pltpu.SemaphoreType.DMA((2,)),
            pltpu.SemaphoreType.DMA((2,)),
        ],
    )
    return run(xf, pf).reshape(B, S, D)


_BS = 2048


def _add_body(x_ref, p_ref, o_ref):
    o_ref[...] = x_ref[...] + p_ref[...]


def _tc_add(x, pos_table):
    B, S, D = x.shape
    bs = min(_BS, S)
    grid = (S // bs, B)
    return pl.pallas_call(
        _add_body,
        grid=grid,
        in_specs=[
            pl.BlockSpec((1, bs, D), lambda i, b: (b, i, 0)),
            pl.BlockSpec((bs, D), lambda i, b: (i, 0)),
        ],
        out_specs=pl.BlockSpec((1, bs, D), lambda i, b: (b, i, 0)),
        out_shape=jax.ShapeDtypeStruct(x.shape, x.dtype),
    )(x, pos_table)


def kernel(x, pos_table):
    return _sc_add(x, pos_table)


# speedup vs baseline: 1.0000x; 1.0000x over previous
import functools

import jax
import jax.numpy as jnp
from jax import lax
from jax.experimental import pallas as pl
from jax.experimental.pallas import tpu as pltpu
from jax.experimental.pallas import tpu_sc as plsc

_NC, _NS = 2, 16
_NW = _NC * _NS
_CHUNK = 16 * 1024
_LANES = 16


_ROWS = 8
_D = 1024
_B = 4


def _sc_body(x_hbm, p_hbm, o_hbm, xb, pb, sx, sp, so):
    S = p_hbm.shape[0]
    spw = S // _NW
    nstep = spw // _ROWS
    wid = lax.axis_index("s") * _NC + lax.axis_index("c")
    s_base = wid * spw

    def pos_copy(step, slot):
        s0 = s_base + step * _ROWS
        return pltpu.make_async_copy(
            p_hbm.at[pl.ds(s0, _ROWS)], pb.at[slot], sp.at[slot])

    def x_copy(step, slot, b):
        s0 = s_base + step * _ROWS
        return pltpu.make_async_copy(
            x_hbm.at[pl.ds(b * S + s0, _ROWS)], xb.at[slot, b], sx.at[slot, b])

    def out_copy(step, slot, b):
        s0 = s_base + step * _ROWS
        return pltpu.make_async_copy(
            xb.at[slot, b], o_hbm.at[pl.ds(b * S + s0, _ROWS)], so.at[slot, b])

    def compute_batch(slot, b):
        @plsc.parallel_loop(0, _ROWS * _D, step=_LANES, unroll=8)
        def _(off):
            r = off // _D
            c = off % _D
            pv = pb[slot, r, pl.ds(c, _LANES)]
            plsc.addupdate(xb.at[slot, b, r, pl.ds(c, _LANES)], pv)

    pos_copy(0, 0).start()
    for b in range(_B):
        x_copy(0, 0, b).start()
    for s in range(nstep):
        slot = s % 2
        if s + 1 < nstep:
            pos_copy(s + 1, 1 - slot).start()
            for b in range(_B):
                if s >= 1:
                    out_copy(s - 1, 1 - slot, b).wait()
                x_copy(s + 1, 1 - slot, b).start()
        pos_copy(s, slot).wait()
        for b in range(_B):
            x_copy(s, slot, b).wait()
            compute_batch(slot, b)
            out_copy(s, slot, b).start()
    for s in (nstep - 2, nstep - 1):
        if s >= 0:
            for b in range(_B):
                out_copy(s, s % 2, b).wait()


def _sc_add(x, pos_table):
    B, S, D = x.shape
    xf = x.reshape(B * S, D)
    pf = pos_table
    run = pl.kernel(
        _sc_body,
        out_type=jax.ShapeDtypeStruct((B * S, D), x.dtype),
        mesh=plsc.VectorSubcoreMesh(
            core_axis_name="c", subcore_axis_name="s",
            num_cores=_NC, num_subcores=_NS,
        ),
        scratch_types=[
            pltpu.VMEM((2, _B, _ROWS, _D), jnp.float32),
            pltpu.VMEM((2, _ROWS, _D), jnp.float32),
            pltpu.SemaphoreType.DMA((2, _B)),
            pltpu.SemaphoreType.DMA((2,)),
            pltpu.SemaphoreType.DMA((2, _B)),
        ],
    )
    return run(xf, pf).reshape(B, S, D)


_BS = 2048


def _add_body(x_ref, p_ref, o_ref):
    o_ref[...] = x_ref[...] + p_ref[...]


def _tc_add(x, pos_table):
    B, S, D = x.shape
    bs = min(_BS, S)
    grid = (S // bs, B)
    return pl.pallas_call(
        _add_body,
        grid=grid,
        in_specs=[
            pl.BlockSpec((1, bs, D), lambda i, b: (b, i, 0)),
            pl.BlockSpec((bs, D), lambda i, b: (i, 0)),
        ],
        out_specs=pl.BlockSpec((1, bs, D), lambda i, b: (b, i, 0)),
        out_shape=jax.ShapeDtypeStruct(x.shape, x.dtype),
    )(x, pos_table)


def kernel(x, pos_table):
    return _sc_add(x, pos_table)


# speedup vs baseline: 1.0015x; 1.0015x over previous
import functools

import jax
import jax.numpy as jnp
from jax import lax
from jax.experimental import pallas as pl
from jax.experimental.pallas import tpu as pltpu
from jax.experimental.pallas import tpu_sc as plsc

_NC, _NS = 2, 16
_NW = _NC * _NS
_CHUNK = 16 * 1024
_LANES = 16


_ROWS = 8
_D = 1024
_B = 4


def _sc_body(x_hbm, p_hbm, o_hbm, xb, pb, sx, sp, so):
    S = p_hbm.shape[0]
    spw = S // _NW
    nstep = spw // _ROWS
    wid = lax.axis_index("s") * _NC + lax.axis_index("c")
    s_base = wid * spw

    def pos_copy(step, slot):
        s0 = s_base + step * _ROWS
        return pltpu.make_async_copy(
            p_hbm.at[pl.ds(s0, _ROWS)], pb.at[slot], sp.at[slot])

    def x_copy(step, slot, b):
        s0 = s_base + step * _ROWS
        return pltpu.make_async_copy(
            x_hbm.at[pl.ds(b * S + s0, _ROWS)], xb.at[slot, b], sx.at[slot, b])

    def out_copy(step, slot, b):
        s0 = s_base + step * _ROWS
        return pltpu.make_async_copy(
            xb.at[slot, b], o_hbm.at[pl.ds(b * S + s0, _ROWS)], so.at[slot, b])

    def compute_batch(slot, b):
        @plsc.parallel_loop(0, _ROWS * _D, step=_LANES, unroll=8)
        def _(off):
            r = off // _D
            c = off % _D
            pv = pb[slot, r, pl.ds(c, _LANES)]
            plsc.addupdate(xb.at[slot, b, r, pl.ds(c, _LANES)], pv)

    for s0 in (0, 1):
        pos_copy(s0, s0).start()
        for b in range(_B):
            x_copy(s0, s0, b).start()
    for s in range(nstep):
        slot = s % 3
        if s + 2 < nstep:
            nslot = (s + 2) % 3
            pos_copy(s + 2, nslot).start()
            for b in range(_B):
                if s >= 1:
                    out_copy(s - 1, nslot, b).wait()
                x_copy(s + 2, nslot, b).start()
        pos_copy(s, slot).wait()
        for b in range(_B):
            x_copy(s, slot, b).wait()
            compute_batch(slot, b)
            out_copy(s, slot, b).start()
    for s in (nstep - 3, nstep - 2, nstep - 1):
        if s >= 0:
            for b in range(_B):
                out_copy(s, s % 3, b).wait()


def _sc_add(x, pos_table):
    B, S, D = x.shape
    xf = x.reshape(B * S, D)
    pf = pos_table
    run = pl.kernel(
        _sc_body,
        out_type=jax.ShapeDtypeStruct((B * S, D), x.dtype),
        mesh=plsc.VectorSubcoreMesh(
            core_axis_name="c", subcore_axis_name="s",
            num_cores=_NC, num_subcores=_NS,
        ),
        scratch_types=[
            pltpu.VMEM((3, _B, _ROWS, _D), jnp.float32),
            pltpu.VMEM((3, _ROWS, _D), jnp.float32),
            pltpu.SemaphoreType.DMA((3, _B)),
            pltpu.SemaphoreType.DMA((3,)),
            pltpu.SemaphoreType.DMA((3, _B)),
        ],
    )
    return run(xf, pf).reshape(B, S, D)


_BS = 2048


def _add_body(x_ref, p_ref, o_ref):
    o_ref[...] = x_ref[...] + p_ref[...]


def _tc_add(x, pos_table):
    B, S, D = x.shape
    bs = min(_BS, S)
    grid = (S // bs, B)
    return pl.pallas_call(
        _add_body,
        grid=grid,
        in_specs=[
            pl.BlockSpec((1, bs, D), lambda i, b: (b, i, 0)),
            pl.BlockSpec((bs, D), lambda i, b: (i, 0)),
        ],
        out_specs=pl.BlockSpec((1, bs, D), lambda i, b: (b, i, 0)),
        out_shape=jax.ShapeDtypeStruct(x.shape, x.dtype),
    )(x, pos_table)


def kernel(x, pos_table):
    return _sc_add(x, pos_table)


# speedup vs baseline: 1.0036x; 1.0021x over previous
import functools

import jax
import jax.numpy as jnp
from jax import lax
from jax.experimental import pallas as pl
from jax.experimental.pallas import tpu as pltpu
from jax.experimental.pallas import tpu_sc as plsc

_NC, _NS = 2, 16
_NW = _NC * _NS
_CHUNK = 16 * 1024
_LANES = 16


_ROWS = 8
_D = 1024
_B = 4


def _sc_body(x_hbm, p_hbm, o_hbm, xb, pb, sx, sp, so, *, s_full, s_cover):
    spw = s_cover // _NW
    nstep = spw // _ROWS
    wid = lax.axis_index("s") * _NC + lax.axis_index("c")
    s_base = wid * spw

    def pos_copy(step, slot):
        s0 = s_base + step * _ROWS
        return pltpu.make_async_copy(
            p_hbm.at[pl.ds(s0, _ROWS)], pb.at[slot], sp.at[slot])

    def x_copy(step, slot, b):
        s0 = s_base + step * _ROWS
        return pltpu.make_async_copy(
            x_hbm.at[pl.ds(b * s_full + s0, _ROWS)], xb.at[slot, b], sx.at[slot, b])

    def out_copy(step, slot, b):
        s0 = s_base + step * _ROWS
        return pltpu.make_async_copy(
            xb.at[slot, b], o_hbm.at[pl.ds(b * s_cover + s0, _ROWS)], so.at[slot, b])

    def compute_batch(slot, b):
        @plsc.parallel_loop(0, _ROWS * _D, step=_LANES, unroll=8)
        def _(off):
            r = off // _D
            c = off % _D
            pv = pb[slot, r, pl.ds(c, _LANES)]
            plsc.addupdate(xb.at[slot, b, r, pl.ds(c, _LANES)], pv)

    for s0 in (0, 1):
        pos_copy(s0, s0).start()
        for b in range(_B):
            x_copy(s0, s0, b).start()
    for s in range(nstep):
        slot = s % 3
        if s + 2 < nstep:
            nslot = (s + 2) % 3
            pos_copy(s + 2, nslot).start()
            for b in range(_B):
                if s >= 1:
                    out_copy(s - 1, nslot, b).wait()
                x_copy(s + 2, nslot, b).start()
        pos_copy(s, slot).wait()
        for b in range(_B):
            x_copy(s, slot, b).wait()
            compute_batch(slot, b)
            out_copy(s, slot, b).start()
    for s in (nstep - 3, nstep - 2, nstep - 1):
        if s >= 0:
            for b in range(_B):
                out_copy(s, s % 3, b).wait()


def _sc_add(x, pos_table, s_cover=None):
    B, S, D = x.shape
    if s_cover is None:
        s_cover = S
    xf = x.reshape(B * S, D)
    pf = pos_table
    run = pl.kernel(
        functools.partial(_sc_body, s_full=S, s_cover=s_cover),
        out_type=jax.ShapeDtypeStruct((B * s_cover, D), x.dtype),
        mesh=plsc.VectorSubcoreMesh(
            core_axis_name="c", subcore_axis_name="s",
            num_cores=_NC, num_subcores=_NS,
        ),
        scratch_types=[
            pltpu.VMEM((3, _B, _ROWS, _D), jnp.float32),
            pltpu.VMEM((3, _ROWS, _D), jnp.float32),
            pltpu.SemaphoreType.DMA((3, _B)),
            pltpu.SemaphoreType.DMA((3,)),
            pltpu.SemaphoreType.DMA((3, _B)),
        ],
    )
    return run(xf, pf).reshape(B, s_cover, D)


_BS = 2048


def _add_body(x_ref, p_ref, o_ref):
    o_ref[...] = x_ref[...] + p_ref[...]


def _tc_add(x, pos_table, s_skip=0):
    B, S, D = x.shape
    bs = min(_BS, S - s_skip)
    grid = ((S - s_skip) // bs, B)
    off = s_skip // bs
    return pl.pallas_call(
        _add_body,
        grid=grid,
        in_specs=[
            pl.BlockSpec((1, bs, D), lambda i, b: (b, i + off, 0)),
            pl.BlockSpec((bs, D), lambda i, b: (i + off, 0)),
        ],
        out_specs=pl.BlockSpec((1, bs, D), lambda i, b: (b, i + off, 0)),
        out_shape=jax.ShapeDtypeStruct(x.shape, x.dtype),
    )(x, pos_table)


_SC_FRAC = 4


def kernel(x, pos_table):
    B, S, D = x.shape
    s_cover = S // _SC_FRAC
    sc_head = _sc_add(x, pos_table, s_cover)
    tc_out = _tc_add(x, pos_table, s_skip=s_cover)
    return lax.dynamic_update_slice(tc_out, sc_head, (0, 0, 0))


# speedup vs baseline: 1.2176x; 1.2132x over previous
import functools

import jax
import jax.numpy as jnp
from jax import lax
from jax.experimental import pallas as pl
from jax.experimental.pallas import tpu as pltpu
from jax.experimental.pallas import tpu_sc as plsc

_NC, _NS = 2, 16
_NW = _NC * _NS
_CHUNK = 16 * 1024
_LANES = 16


_ROWS = 8
_D = 1024
_B = 4


def _sc_body(x_hbm, p_hbm, o_hbm, xb, pb, sx, sp, so, *, s_full, s_cover):
    spw = s_cover // _NW
    nstep = spw // _ROWS
    wid = lax.axis_index("s") * _NC + lax.axis_index("c")
    s_base = wid * spw

    def pos_copy(step, slot):
        s0 = s_base + step * _ROWS
        return pltpu.make_async_copy(
            p_hbm.at[pl.ds(s0, _ROWS)], pb.at[slot], sp.at[slot])

    def x_copy(step, slot, b):
        s0 = s_base + step * _ROWS
        return pltpu.make_async_copy(
            x_hbm.at[pl.ds(b * s_full + s0, _ROWS)], xb.at[slot, b], sx.at[slot, b])

    def out_copy(step, slot, b):
        s0 = s_base + step * _ROWS
        return pltpu.make_async_copy(
            xb.at[slot, b], o_hbm.at[pl.ds(b * s_cover + s0, _ROWS)], so.at[slot, b])

    def compute_batch(slot, b):
        @plsc.parallel_loop(0, _ROWS * _D, step=_LANES, unroll=8)
        def _(off):
            r = off // _D
            c = off % _D
            pv = pb[slot, r, pl.ds(c, _LANES)]
            plsc.addupdate(xb.at[slot, b, r, pl.ds(c, _LANES)], pv)

    for s0 in (0, 1):
        pos_copy(s0, s0).start()
        for b in range(_B):
            x_copy(s0, s0, b).start()
    for s in range(nstep):
        slot = s % 3
        if s + 2 < nstep:
            nslot = (s + 2) % 3
            pos_copy(s + 2, nslot).start()
            for b in range(_B):
                if s >= 1:
                    out_copy(s - 1, nslot, b).wait()
                x_copy(s + 2, nslot, b).start()
        pos_copy(s, slot).wait()
        for b in range(_B):
            x_copy(s, slot, b).wait()
            compute_batch(slot, b)
            out_copy(s, slot, b).start()
    for s in (nstep - 3, nstep - 2, nstep - 1):
        if s >= 0:
            for b in range(_B):
                out_copy(s, s % 3, b).wait()


def _sc_add(x, pos_table, s_cover=None):
    B, S, D = x.shape
    if s_cover is None:
        s_cover = S
    xf = x.reshape(B * S, D)
    pf = pos_table
    run = pl.kernel(
        functools.partial(_sc_body, s_full=S, s_cover=s_cover),
        out_type=jax.ShapeDtypeStruct((B * s_cover, D), x.dtype),
        mesh=plsc.VectorSubcoreMesh(
            core_axis_name="c", subcore_axis_name="s",
            num_cores=_NC, num_subcores=_NS,
        ),
        scratch_types=[
            pltpu.VMEM((3, _B, _ROWS, _D), jnp.float32),
            pltpu.VMEM((3, _ROWS, _D), jnp.float32),
            pltpu.SemaphoreType.DMA((3, _B)),
            pltpu.SemaphoreType.DMA((3,)),
            pltpu.SemaphoreType.DMA((3, _B)),
        ],
    )
    return run(xf, pf).reshape(B, s_cover, D)


_BS = 2048


def _add_body(x_ref, p_ref, o_ref):
    o_ref[...] = x_ref[...] + p_ref[...]


def _tc_add(x, pos_table, s_skip=0):
    B, S, D = x.shape
    bs = min(_BS, S - s_skip)
    grid = ((S - s_skip) // bs, B)
    off = s_skip // bs
    return pl.pallas_call(
        _add_body,
        grid=grid,
        in_specs=[
            pl.BlockSpec((1, bs, D), lambda i, b: (b, i + off, 0)),
            pl.BlockSpec((bs, D), lambda i, b: (i + off, 0)),
        ],
        out_specs=pl.BlockSpec((1, bs, D), lambda i, b: (b, i + off, 0)),
        out_shape=jax.ShapeDtypeStruct(x.shape, x.dtype),
    )(x, pos_table)


_SC_FRAC = 8


def kernel(x, pos_table):
    B, S, D = x.shape
    s_cover = S // _SC_FRAC
    sc_head = _sc_add(x, pos_table, s_cover)
    tc_out = _tc_add(x, pos_table, s_skip=s_cover)
    return lax.dynamic_update_slice(tc_out, sc_head, (0, 0, 0))
